# baseline (speedup 1.0000x reference)
import functools

import jax
import jax.numpy as jnp
from jax import lax
from jax.experimental import pallas as pl
from jax.experimental.pallas import tpu as pltpu
from jax.experimental.pallas import tpu_sc as plsc

NC = 2
NS = 16
NW = NC * NS
CH = 128


@functools.cache
def _build(B, dims, nch, tc_tiling):
    nt = len(dims)
    mesh = plsc.VectorSubcoreMesh(core_axis_name="c", subcore_axis_name="s")
    bpw = B // NW
    nr = nch // 2

    @functools.partial(
        pl.kernel,
        out_type=tuple(jax.ShapeDtypeStruct((B, d), jnp.float32) for d in dims),
        mesh=mesh,
        compiler_params=pltpu.CompilerParams(use_tc_tiling_on_sc=tc_tiling),
        scratch_types=[pltpu.VMEM((2, CH), jnp.int32) for _ in dims]
        + [pltpu.VMEM((2, CH, d), jnp.float32) for d in dims]
        + [pltpu.SemaphoreType.DMA] * 6,
    )
    def k(*refs):
        ids = refs[0:nt]
        tables = refs[nt:2 * nt]
        outs = refs[2 * nt:3 * nt]
        idx_v = refs[3 * nt:4 * nt]
        rows = refs[4 * nt:5 * nt]
        gsem0, gsem1, ssem0, ssem1, isem0, isem1 = refs[5 * nt:]
        gsem = (gsem0, gsem1)
        ssem = (ssem0, ssem1)
        isem = (isem0, isem1)
        wid = lax.axis_index("s") * NC + lax.axis_index("c")
        base = wid * bpw

        def idx_copies(c, s):
            return [(ids[t].at[wid, c], idx_v[t].at[s], isem[s]) for t in range(nt)]

        def gather_copies(s):
            return [(tables[t].at[idx_v[t].at[s]], rows[t].at[s], gsem[s])
                    for t in range(nt)]

        def store_copies(c, s):
            off = base + c * CH
            return [(rows[t].at[s], outs[t].at[pl.ds(off, CH)], ssem[s])
                    for t in range(nt)]

        def issue(copies):
            for src, dst, sem in copies:
                pltpu.async_copy(src, dst, sem)

        def wait(copies):
            for src, dst, sem in copies:
                pltpu.make_async_copy(src, dst, sem).wait()

        issue(idx_copies(0, 0))
        issue(idx_copies(1, 1))
        wait(idx_copies(0, 0))
        issue(gather_copies(0))

        @pl.loop(0, nr)
        def _round(r):
            c0 = 2 * r
            c1 = c0 + 1

            wait(gather_copies(0))
            issue(store_copies(c0, 0))

            @pl.when(r < nr - 1)
            def _():
                issue(idx_copies(c0 + 2, 0))

            @pl.when(r > 0)
            def _():
                wait(store_copies(c0 - 1, 1))
            wait(idx_copies(c1, 1))
            issue(gather_copies(1))

            wait(gather_copies(1))
            issue(store_copies(c1, 1))

            @pl.when(r < nr - 1)
            def _():
                issue(idx_copies(c1 + 2, 1))
                wait(store_copies(c0, 0))
                wait(idx_copies(c1 + 1, 0))
                issue(gather_copies(0))

        wait(store_copies(nch - 2, 0))
        wait(store_copies(nch - 1, 1))

    return k


def kernel(nl_token_ids, code_token_ids, ast_token_ids,
           nl_table, code_table, ast_table):
    Bt, S = nl_token_ids.shape
    B = Bt * S
    assert B % (NW * CH * 2) == 0
    nch = B // (NW * CH)

    k_nl_code = _build(B, (nl_table.shape[1], code_table.shape[1]), nch, True)
    k_ast = _build(B, (ast_table.shape[1],), nch, False)

    nl_ids, code_ids, ast_ids = (x.reshape(NW, nch, CH) for x in
                                 (nl_token_ids, code_token_ids, ast_token_ids))
    nl_out, code_out = k_nl_code(nl_ids, code_ids, nl_table, code_table)
    (ast_out,) = k_ast(ast_ids, ast_table)
    return (nl_out.reshape(Bt, S, -1),
            code_out.reshape(Bt, S, -1),
            ast_out.reshape(Bt, S, -1))
